# Initial kernel scaffold; baseline (speedup 1.0000x reference)
import jax
import jax.numpy as jnp
from jax.experimental import pallas as pl


def kernel(inputs, embeddings):
    raise NotImplementedError("write your pallas kernel here")


# speedup vs baseline: 1.5336x; 1.5336x over previous
import functools

import jax
import jax.numpy as jnp
from jax import lax
from jax.experimental import pallas as pl
from jax.experimental.pallas import tpu as pltpu
from jax.experimental.pallas import tpu_sc as plsc

_NUM_CODES = 8192
_CODE_DIM = 256
_TOKENS = 16384
_T_BLK = 512
_C_BLK = 2048
_N_CHUNKS = _NUM_CODES // _C_BLK


def _argmin_body(x_ref, e_ref, l2x_ref, l2e_ref, codes_ref):
    x = x_ref[...]
    l2x = l2x_ref[...]

    best_val = jnp.full((_T_BLK, 1), jnp.inf, dtype=jnp.float32)
    best_idx = jnp.zeros((_T_BLK, 1), dtype=jnp.int32)

    for c in range(_N_CHUNKS):
        ec = e_ref[pl.ds(c * _C_BLK, _C_BLK), :]
        l2e = l2e_ref[:, pl.ds(c * _C_BLK, _C_BLK)]
        dot = lax.dot_general(
            x, ec, (((1,), (1,)), ((), ())),
            preferred_element_type=jnp.float32)
        dist = (l2x + l2e) - 2.0 * dot
        m = jnp.min(dist, axis=1, keepdims=True)
        iota = lax.broadcasted_iota(jnp.int32, (_T_BLK, _C_BLK), 1)
        idx = jnp.min(
            jnp.where(dist <= m, iota + c * _C_BLK, _NUM_CODES),
            axis=1, keepdims=True)
        take = m < best_val
        best_val = jnp.where(take, m, best_val)
        best_idx = jnp.where(take, idx, best_idx)

    codes_ref[...] = best_idx


def _tc_codes(x, e, l2x, l2e):
    return pl.pallas_call(
        _argmin_body,
        grid=(_TOKENS // _T_BLK,),
        in_specs=[
            pl.BlockSpec((_T_BLK, _CODE_DIM), lambda i: (i, 0)),
            pl.BlockSpec((_NUM_CODES, _CODE_DIM), lambda i: (0, 0)),
            pl.BlockSpec((_T_BLK, 1), lambda i: (i, 0)),
            pl.BlockSpec((1, _NUM_CODES), lambda i: (0, 0)),
        ],
        out_specs=pl.BlockSpec((_T_BLK, 1), lambda i: (i, 0)),
        out_shape=jax.ShapeDtypeStruct((_TOKENS, 1), jnp.int32),
    )(x, e, l2x, l2e)


_NW = 32
_ROWS_PER_W = _TOKENS // _NW
_G_CHUNK = 128


@functools.cache
def _sc_gather_kernel():
    @functools.partial(
        pl.kernel,
        out_type=jax.ShapeDtypeStruct((_TOKENS, _CODE_DIM), jnp.float32),
        mesh=plsc.VectorSubcoreMesh(
            core_axis_name="c", subcore_axis_name="s",
            num_cores=2, num_subcores=16),
        scratch_types=[
            pltpu.VMEM((_G_CHUNK,), jnp.int32),
            pltpu.VMEM((_G_CHUNK, _CODE_DIM), jnp.float32),
            pltpu.SemaphoreType.DMA,
        ],
    )
    def _sc_gather(table_hbm, idx_hbm, out_hbm, idx_v, rows_v, sem):
        wid = lax.axis_index("s") * 2 + lax.axis_index("c")
        base = wid * _ROWS_PER_W
        for g in range(_ROWS_PER_W // _G_CHUNK):
            off = base + g * _G_CHUNK
            pltpu.sync_copy(idx_hbm.at[pl.ds(off, _G_CHUNK)], idx_v)
            pltpu.async_copy(table_hbm.at[idx_v], rows_v, sem).wait()
            pltpu.sync_copy(rows_v, out_hbm.at[pl.ds(off, _G_CHUNK)])

    return _sc_gather


def kernel(inputs, embeddings):
    x = inputs.reshape(_TOKENS, _CODE_DIM)
    l2x = jnp.sum(inputs ** 2, axis=-1, keepdims=True).reshape(_TOKENS, 1)
    l2e = jnp.sum(embeddings ** 2, axis=-1).reshape(1, _NUM_CODES)
    codes = _tc_codes(x, embeddings, l2x, l2e)
    out = _sc_gather_kernel()(embeddings, codes.reshape(_TOKENS))
    return out.reshape(inputs.shape)


# speedup vs baseline: 1.6955x; 1.1056x over previous
import functools

import jax
import jax.numpy as jnp
from jax import lax
from jax.experimental import pallas as pl
from jax.experimental.pallas import tpu as pltpu
from jax.experimental.pallas import tpu_sc as plsc

_NUM_CODES = 8192
_CODE_DIM = 256
_TOKENS = 16384
_T_BLK = 512
_C_BLK = 2048
_N_CHUNKS = _NUM_CODES // _C_BLK


def _argmin_body(x_ref, e_ref, l2x_ref, l2e_ref, codes_ref):
    xm2 = -2.0 * x_ref[...]
    l2x = l2x_ref[...]

    best_val = jnp.full((_T_BLK, 1), jnp.inf, dtype=jnp.float32)
    best_idx = jnp.full((_T_BLK, 1), 0.0, dtype=jnp.float32)

    for c in range(_N_CHUNKS):
        ec = e_ref[pl.ds(c * _C_BLK, _C_BLK), :]
        l2e = l2e_ref[:, pl.ds(c * _C_BLK, _C_BLK)]
        nd2 = lax.dot_general(
            xm2, ec, (((1,), (1,)), ((), ())),
            preferred_element_type=jnp.float32)
        dist = (l2x + l2e) + nd2
        m = jnp.min(dist, axis=1, keepdims=True)
        iota = lax.broadcasted_iota(
            jnp.int32, (_T_BLK, _C_BLK), 1).astype(jnp.float32)
        idx = jnp.min(
            jnp.where(dist <= m, iota, float(_C_BLK)),
            axis=1, keepdims=True)
        take = m < best_val
        best_val = jnp.where(take, m, best_val)
        best_idx = jnp.where(take, idx + float(c * _C_BLK), best_idx)

    codes_ref[...] = best_idx.astype(jnp.int32)


def _tc_codes(x, e, l2x, l2e):
    return pl.pallas_call(
        _argmin_body,
        grid=(_TOKENS // _T_BLK,),
        in_specs=[
            pl.BlockSpec((_T_BLK, _CODE_DIM), lambda i: (i, 0)),
            pl.BlockSpec((_NUM_CODES, _CODE_DIM), lambda i: (0, 0)),
            pl.BlockSpec((_T_BLK, 1), lambda i: (i, 0)),
            pl.BlockSpec((1, _NUM_CODES), lambda i: (0, 0)),
        ],
        out_specs=pl.BlockSpec((_T_BLK, 1), lambda i: (i, 0)),
        out_shape=jax.ShapeDtypeStruct((_TOKENS, 1), jnp.int32),
    )(x, e, l2x, l2e)


_NW = 32
_ROWS_PER_W = _TOKENS // _NW
_G_CHUNK = 128


@functools.cache
def _sc_gather_kernel():
    @functools.partial(
        pl.kernel,
        out_type=jax.ShapeDtypeStruct((_TOKENS, _CODE_DIM), jnp.float32),
        mesh=plsc.VectorSubcoreMesh(
            core_axis_name="c", subcore_axis_name="s",
            num_cores=2, num_subcores=16),
        scratch_types=[
            pltpu.VMEM((_G_CHUNK,), jnp.int32),
            pltpu.VMEM((_G_CHUNK, _CODE_DIM), jnp.float32),
            pltpu.SemaphoreType.DMA,
        ],
    )
    def _sc_gather(table_hbm, idx_hbm, out_hbm, idx_v, rows_v, sem):
        wid = lax.axis_index("s") * 2 + lax.axis_index("c")
        base = wid * _ROWS_PER_W
        for g in range(_ROWS_PER_W // _G_CHUNK):
            off = base + g * _G_CHUNK
            pltpu.sync_copy(idx_hbm.at[pl.ds(off, _G_CHUNK)], idx_v)
            pltpu.async_copy(table_hbm.at[idx_v], rows_v, sem).wait()
            pltpu.sync_copy(rows_v, out_hbm.at[pl.ds(off, _G_CHUNK)])

    return _sc_gather


def kernel(inputs, embeddings):
    x = inputs.reshape(_TOKENS, _CODE_DIM)
    l2x = jnp.sum(inputs ** 2, axis=-1, keepdims=True).reshape(_TOKENS, 1)
    l2e = jnp.sum(embeddings ** 2, axis=-1).reshape(1, _NUM_CODES)
    codes = _tc_codes(x, embeddings, l2x, l2e)
    out = _sc_gather_kernel()(embeddings, codes.reshape(_TOKENS))
    return out.reshape(inputs.shape)


# speedup vs baseline: 1.7393x; 1.0258x over previous
import functools

import jax
import jax.numpy as jnp
from jax import lax
from jax.experimental import pallas as pl
from jax.experimental.pallas import tpu as pltpu
from jax.experimental.pallas import tpu_sc as plsc

_NUM_CODES = 8192
_CODE_DIM = 256
_TOKENS = 16384
_T_BLK = 1024
_C_BLK = 2048
_N_CHUNKS = _NUM_CODES // _C_BLK


def _argmin_body(x_ref, e_ref, l2x_ref, l2e_ref, codes_ref):
    xm2 = -2.0 * x_ref[...]
    l2x = l2x_ref[...]

    best_val = jnp.full((_T_BLK, 1), jnp.inf, dtype=jnp.float32)
    best_idx = jnp.full((_T_BLK, 1), 0.0, dtype=jnp.float32)

    for c in range(_N_CHUNKS):
        ec = e_ref[pl.ds(c * _C_BLK, _C_BLK), :]
        l2e = l2e_ref[:, pl.ds(c * _C_BLK, _C_BLK)]
        nd2 = lax.dot_general(
            xm2, ec, (((1,), (1,)), ((), ())),
            preferred_element_type=jnp.float32)
        dist = (l2x + l2e) + nd2
        m = jnp.min(dist, axis=1, keepdims=True)
        iota = lax.broadcasted_iota(
            jnp.int32, (_T_BLK, _C_BLK), 1).astype(jnp.float32)
        idx = jnp.min(
            jnp.where(dist <= m, iota, float(_C_BLK)),
            axis=1, keepdims=True)
        take = m < best_val
        best_val = jnp.where(take, m, best_val)
        best_idx = jnp.where(take, idx + float(c * _C_BLK), best_idx)

    codes_ref[...] = best_idx.astype(jnp.int32)


def _tc_codes(x, e, l2x, l2e):
    return pl.pallas_call(
        _argmin_body,
        grid=(_TOKENS // _T_BLK,),
        in_specs=[
            pl.BlockSpec((_T_BLK, _CODE_DIM), lambda i: (i, 0)),
            pl.BlockSpec((_NUM_CODES, _CODE_DIM), lambda i: (0, 0)),
            pl.BlockSpec((_T_BLK, 1), lambda i: (i, 0)),
            pl.BlockSpec((1, _NUM_CODES), lambda i: (0, 0)),
        ],
        out_specs=pl.BlockSpec((_T_BLK, 1), lambda i: (i, 0)),
        out_shape=jax.ShapeDtypeStruct((_TOKENS, 1), jnp.int32),
    )(x, e, l2x, l2e)


_NW = 32
_ROWS_PER_W = _TOKENS // _NW
_G_CHUNK = 128


@functools.cache
def _sc_gather_kernel():
    @functools.partial(
        pl.kernel,
        out_type=jax.ShapeDtypeStruct((_TOKENS, _CODE_DIM), jnp.float32),
        mesh=plsc.VectorSubcoreMesh(
            core_axis_name="c", subcore_axis_name="s",
            num_cores=2, num_subcores=16),
        scratch_types=[
            pltpu.VMEM((_G_CHUNK,), jnp.int32),
            pltpu.VMEM((_G_CHUNK, _CODE_DIM), jnp.float32),
            pltpu.SemaphoreType.DMA,
        ],
    )
    def _sc_gather(table_hbm, idx_hbm, out_hbm, idx_v, rows_v, sem):
        wid = lax.axis_index("s") * 2 + lax.axis_index("c")
        base = wid * _ROWS_PER_W
        for g in range(_ROWS_PER_W // _G_CHUNK):
            off = base + g * _G_CHUNK
            pltpu.sync_copy(idx_hbm.at[pl.ds(off, _G_CHUNK)], idx_v)
            pltpu.async_copy(table_hbm.at[idx_v], rows_v, sem).wait()
            pltpu.sync_copy(rows_v, out_hbm.at[pl.ds(off, _G_CHUNK)])

    return _sc_gather


def kernel(inputs, embeddings):
    x = inputs.reshape(_TOKENS, _CODE_DIM)
    l2x = jnp.sum(inputs ** 2, axis=-1, keepdims=True).reshape(_TOKENS, 1)
    l2e = jnp.sum(embeddings ** 2, axis=-1).reshape(1, _NUM_CODES)
    codes = _tc_codes(x, embeddings, l2x, l2e)
    out = _sc_gather_kernel()(embeddings, codes.reshape(_TOKENS))
    return out.reshape(inputs.shape)


# speedup vs baseline: 1.8840x; 1.0832x over previous
import functools

import jax
import jax.numpy as jnp
from jax import lax
from jax.experimental import pallas as pl
from jax.experimental.pallas import tpu as pltpu
from jax.experimental.pallas import tpu_sc as plsc

_NUM_CODES = 8192
_CODE_DIM = 256
_TOKENS = 16384
_T_BLK = 1024
_C_BLK = 2048
_N_CHUNKS = _NUM_CODES // _C_BLK


def _argmin_body(x_ref, e_ref, l2e_ref, codes_ref):
    x = x_ref[...]
    xm2 = -2.0 * x
    l2x = jnp.sum(x * x, axis=1, keepdims=True)

    best_val = jnp.full((_T_BLK, 1), jnp.inf, dtype=jnp.float32)
    best_idx = jnp.full((_T_BLK, 1), 0.0, dtype=jnp.float32)

    for c in range(_N_CHUNKS):
        ec = e_ref[pl.ds(c * _C_BLK, _C_BLK), :]
        l2e = l2e_ref[:, pl.ds(c * _C_BLK, _C_BLK)]
        nd2 = lax.dot_general(
            xm2, ec, (((1,), (1,)), ((), ())),
            preferred_element_type=jnp.float32)
        dist = (l2x + l2e) + nd2
        m = jnp.min(dist, axis=1, keepdims=True)
        iota = lax.broadcasted_iota(
            jnp.int32, (_T_BLK, _C_BLK), 1).astype(jnp.float32)
        idx = jnp.min(
            jnp.where(dist <= m, iota, float(_C_BLK)),
            axis=1, keepdims=True)
        take = m < best_val
        best_val = jnp.where(take, m, best_val)
        best_idx = jnp.where(take, idx + float(c * _C_BLK), best_idx)

    codes_ref[...] = best_idx.astype(jnp.int32)


def _tc_codes(x, e, l2e):
    return pl.pallas_call(
        _argmin_body,
        grid=(_TOKENS // _T_BLK,),
        in_specs=[
            pl.BlockSpec((_T_BLK, _CODE_DIM), lambda i: (i, 0)),
            pl.BlockSpec((_NUM_CODES, _CODE_DIM), lambda i: (0, 0)),
            pl.BlockSpec((1, _NUM_CODES), lambda i: (0, 0)),
        ],
        out_specs=pl.BlockSpec((_T_BLK, 1), lambda i: (i, 0)),
        out_shape=jax.ShapeDtypeStruct((_TOKENS, 1), jnp.int32),
    )(x, e, l2e)


_NW = 32
_ROWS_PER_W = _TOKENS // _NW
_G_CHUNK = 128


@functools.cache
def _sc_gather_kernel():
    @functools.partial(
        pl.kernel,
        out_type=jax.ShapeDtypeStruct((_TOKENS, _CODE_DIM), jnp.float32),
        mesh=plsc.VectorSubcoreMesh(
            core_axis_name="c", subcore_axis_name="s",
            num_cores=2, num_subcores=16),
        scratch_types=[
            pltpu.VMEM((_G_CHUNK,), jnp.int32),
            pltpu.VMEM((_G_CHUNK, _CODE_DIM), jnp.float32),
            pltpu.SemaphoreType.DMA,
        ],
    )
    def _sc_gather(table_hbm, idx_hbm, out_hbm, idx_v, rows_v, sem):
        wid = lax.axis_index("s") * 2 + lax.axis_index("c")
        base = wid * _ROWS_PER_W
        for g in range(_ROWS_PER_W // _G_CHUNK):
            off = base + g * _G_CHUNK
            pltpu.sync_copy(idx_hbm.at[pl.ds(off, _G_CHUNK)], idx_v)
            pltpu.async_copy(table_hbm.at[idx_v], rows_v, sem).wait()
            pltpu.sync_copy(rows_v, out_hbm.at[pl.ds(off, _G_CHUNK)])

    return _sc_gather


def kernel(inputs, embeddings):
    x = inputs.reshape(_TOKENS, _CODE_DIM)
    l2e = jnp.sum(embeddings ** 2, axis=-1).reshape(1, _NUM_CODES)
    codes = _tc_codes(x, embeddings, l2e)
    out = _sc_gather_kernel()(embeddings, codes.reshape(_TOKENS))
    return out.reshape(inputs.shape)


# speedup vs baseline: 1.9040x; 1.0107x over previous
import functools

import jax
import jax.numpy as jnp
from jax import lax
from jax.experimental import pallas as pl
from jax.experimental.pallas import tpu as pltpu
from jax.experimental.pallas import tpu_sc as plsc

_NUM_CODES = 8192
_CODE_DIM = 256
_TOKENS = 16384
_T_BLK = 1024
_C_BLK = 2048
_N_CHUNKS = _NUM_CODES // _C_BLK


def _argmin_body(x_ref, e_ref, l2e_ref, codes_ref):
    x = x_ref[...]
    xm2 = -2.0 * x
    l2x = jnp.sum(x * x, axis=1, keepdims=True)

    best_val = jnp.full((_T_BLK, 1), jnp.inf, dtype=jnp.float32)
    best_idx = jnp.full((_T_BLK, 1), 0.0, dtype=jnp.float32)

    for c in range(_N_CHUNKS):
        ec = e_ref[pl.ds(c * _C_BLK, _C_BLK), :]
        l2e = l2e_ref[:, pl.ds(c * _C_BLK, _C_BLK)]
        nd2 = lax.dot_general(
            xm2, ec, (((1,), (1,)), ((), ())),
            preferred_element_type=jnp.float32)
        dist = (l2x + l2e) + nd2
        m = jnp.min(dist, axis=1, keepdims=True)
        iota = lax.broadcasted_iota(
            jnp.int32, (_T_BLK, _C_BLK), 1).astype(jnp.float32)
        idx = jnp.min(
            jnp.where(dist <= m, iota, float(_C_BLK)),
            axis=1, keepdims=True)
        take = m < best_val
        best_val = jnp.where(take, m, best_val)
        best_idx = jnp.where(take, idx + float(c * _C_BLK), best_idx)

    codes_ref[...] = best_idx.astype(jnp.int32)


def _tc_codes(x, e, l2e):
    return pl.pallas_call(
        _argmin_body,
        grid=(_TOKENS // _T_BLK,),
        in_specs=[
            pl.BlockSpec((_T_BLK, _CODE_DIM), lambda i: (i, 0)),
            pl.BlockSpec((_NUM_CODES, _CODE_DIM), lambda i: (0, 0)),
            pl.BlockSpec((1, _NUM_CODES), lambda i: (0, 0)),
        ],
        out_specs=pl.BlockSpec((_T_BLK, 1), lambda i: (i, 0)),
        out_shape=jax.ShapeDtypeStruct((_TOKENS, 1), jnp.int32),
    )(x, e, l2e)


_NW = 32
_ROWS_PER_W = _TOKENS // _NW
_G_CHUNK = 128


@functools.cache
def _sc_gather_kernel():
    @functools.partial(
        pl.kernel,
        out_type=jax.ShapeDtypeStruct((_TOKENS, _CODE_DIM), jnp.float32),
        mesh=plsc.VectorSubcoreMesh(
            core_axis_name="c", subcore_axis_name="s",
            num_cores=2, num_subcores=16),
        scratch_types=[
            pltpu.VMEM((_ROWS_PER_W,), jnp.int32),
            pltpu.VMEM((_G_CHUNK, _CODE_DIM), jnp.float32),
            pltpu.VMEM((_G_CHUNK, _CODE_DIM), jnp.float32),
            pltpu.SemaphoreType.DMA,
            pltpu.SemaphoreType.DMA,
            pltpu.SemaphoreType.DMA,
            pltpu.SemaphoreType.DMA,
        ],
    )
    def _sc_gather(table_hbm, idx_hbm, out_hbm,
                   idx_all, rows0, rows1, gs0, gs1, os0, os1):
        wid = lax.axis_index("s") * 2 + lax.axis_index("c")
        base = wid * _ROWS_PER_W
        n = _ROWS_PER_W // _G_CHUNK
        rows, gs, os = [rows0, rows1], [gs0, gs1], [os0, os1]
        pltpu.sync_copy(idx_hbm.at[pl.ds(base, _ROWS_PER_W)], idx_all)
        gh = [
            pltpu.async_copy(
                table_hbm.at[idx_all.at[pl.ds(g * _G_CHUNK, _G_CHUNK)]],
                rows[g], gs[g])
            for g in range(2)
        ]
        oh = [None, None]
        for g in range(n):
            b = g % 2
            gh[b].wait()
            oh[b] = pltpu.async_copy(
                rows[b], out_hbm.at[pl.ds(base + g * _G_CHUNK, _G_CHUNK)],
                os[b])
            if g + 2 < n:
                oh[b].wait()
                gh[b] = pltpu.async_copy(
                    table_hbm.at[
                        idx_all.at[pl.ds((g + 2) * _G_CHUNK, _G_CHUNK)]],
                    rows[b], gs[b])
        oh[0].wait()
        oh[1].wait()

    return _sc_gather


def kernel(inputs, embeddings):
    x = inputs.reshape(_TOKENS, _CODE_DIM)
    l2e = jnp.sum(embeddings ** 2, axis=-1).reshape(1, _NUM_CODES)
    codes = _tc_codes(x, embeddings, l2e)
    out = _sc_gather_kernel()(embeddings, codes.reshape(_TOKENS))
    return out.reshape(inputs.shape)
